# baseline (speedup 1.0000x reference)
import functools

import jax
import jax.numpy as jnp
from jax import lax
from jax.experimental import pallas as pl
from jax.experimental.pallas import tpu as pltpu
from jax.experimental.pallas import tpu_sc as plsc

_N = 100000
_NUM_SP = 1024
_LANE = 128
_TILES = 782
_OUT_FLAT = _TILES * 4 * _LANE
_NUM_CORES = 2
_NW = 32
_TPW = 25
_CHUNK = _TPW * _LANE
_WSTAGE = _TPW * 4 * _LANE
_LAST_BASE = _TILES * _LANE - _CHUNK
_LAST_VALID = _N - _LAST_BASE


def _gather_body(tab_t, tab_r, idx, out_t, out_r,
                 tab_t_v, tab_r_v, idx_v, out_t_v, out_r_v, sem):
    wid = lax.axis_index("s") * _NUM_CORES + lax.axis_index("c")
    is_tail = wid == _NW - 1
    base = jnp.where(is_tail, _LAST_BASE, wid * _CHUNK)

    ct = pltpu.async_copy(tab_t, tab_t_v, sem)
    cr = pltpu.async_copy(tab_r, tab_r_v, sem)

    @pl.when(jnp.logical_not(is_tail))
    def _():
        pltpu.sync_copy(idx.at[pl.ds(base, _CHUNK)], idx_v.at[pl.ds(0, _CHUNK)])

    @pl.when(is_tail)
    def _():
        pltpu.sync_copy(idx.at[pl.ds(_LAST_BASE, _LAST_VALID)],
                        idx_v.at[pl.ds(0, _LAST_VALID)])
        zeros = jnp.zeros((16,), jnp.int32)
        for z in range(_LAST_VALID, _CHUNK, 16):
            idx_v[pl.ds(z, 16)] = zeros

    ct.wait()
    cr.wait()

    def tile_body(t, carry):
        tbase = t * 512
        gbase = t * 128
        for h in range(1):
            sps = [idx_v[pl.ds(gbase + (h * 8 + s) * 16, 16)]
                   for s in range(8)]
            addrs = [sp + (sp >> 7) * 384 for sp in sps]
            vals = [plsc.load_gather(tab_v, [a + c * 128])
                    for a in addrs
                    for tab_v in (tab_t_v, tab_r_v)
                    for c in range(3)]
            i = 0
            for s in range(8):
                off = tbase + (h * 8 + s) * 16
                for out_v in (out_t_v, out_r_v):
                    for c in range(3):
                        out_v[pl.ds(off + c * 128, 16)] = vals[i]
                        i += 1
        return carry

    lax.fori_loop(0, _TPW, tile_body, 0)

    obase = base * 4
    pltpu.sync_copy(out_t_v, out_t.at[pl.ds(obase, _WSTAGE)])
    pltpu.sync_copy(out_r_v, out_r.at[pl.ds(obase, _WSTAGE)])


def _tilebytes(tab):
    p = jnp.pad(tab, ((0, 0), (0, 1)))
    return p.reshape(8, 128, 4).transpose(0, 2, 1).reshape(-1)


def _untile(flat):
    tiles = flat.reshape(_TILES, 4, _LANE)
    return tiles.transpose(0, 2, 1).reshape(_TILES * _LANE, 4)[:_N, :3]


def kernel(sp_delta_t, sp_delta_r, p2sp):
    mesh = plsc.VectorSubcoreMesh(core_axis_name="c", subcore_axis_name="s")
    run = pl.kernel(
        _gather_body,
        mesh=mesh,
        compiler_params=pltpu.CompilerParams(needs_layout_passes=False),
        out_type=(
            jax.ShapeDtypeStruct((_OUT_FLAT,), jnp.float32),
            jax.ShapeDtypeStruct((_OUT_FLAT,), jnp.float32),
        ),
        scratch_types=[
            pltpu.VMEM((_NUM_SP * 4,), jnp.float32),
            pltpu.VMEM((_NUM_SP * 4,), jnp.float32),
            pltpu.VMEM((_CHUNK,), jnp.int32),
            pltpu.VMEM((_WSTAGE,), jnp.float32),
            pltpu.VMEM((_WSTAGE,), jnp.float32),
            pltpu.SemaphoreType.DMA,
        ],
    )
    flat_t, flat_r = run(_tilebytes(sp_delta_t), _tilebytes(sp_delta_r), p2sp)
    return _untile(flat_t), _untile(flat_r)


# speedup vs baseline: 1.0007x; 1.0007x over previous
import functools

import jax
import jax.numpy as jnp
from jax import lax
from jax.experimental import pallas as pl
from jax.experimental.pallas import tpu as pltpu
from jax.experimental.pallas import tpu_sc as plsc

_N = 100000
_NUM_SP = 1024
_LANE = 128
_TILES = 782
_OUT_FLAT = _TILES * 4 * _LANE
_NUM_CORES = 2
_NW = 32
_TPW = 25
_CHUNK = _TPW * _LANE
_WSTAGE = _TPW * 4 * _LANE
_LAST_BASE = _TILES * _LANE - _CHUNK
_LAST_VALID = _N - _LAST_BASE


def _gather_body(tab, idx, out_t, out_r,
                 tab_v, idx_v, out_t_v, out_r_v, sem):
    wid = lax.axis_index("s") * _NUM_CORES + lax.axis_index("c")
    is_tail = wid == _NW - 1
    base = jnp.where(is_tail, _LAST_BASE, wid * _CHUNK)

    ctab = pltpu.async_copy(tab, tab_v, sem)

    @pl.when(jnp.logical_not(is_tail))
    def _():
        pltpu.sync_copy(idx.at[pl.ds(base, _CHUNK)], idx_v.at[pl.ds(0, _CHUNK)])

    @pl.when(is_tail)
    def _():
        pltpu.sync_copy(idx.at[pl.ds(_LAST_BASE, _LAST_VALID)],
                        idx_v.at[pl.ds(0, _LAST_VALID)])
        zeros = jnp.zeros((16,), jnp.int32)
        for z in range(_LAST_VALID, _CHUNK, 16):
            idx_v[pl.ds(z, 16)] = zeros

    ctab.wait()

    def tile_body(t, carry):
        tbase = t * 512
        gbase = t * 128
        for h in range(1):
            sps = [idx_v[pl.ds(gbase + (h * 8 + s) * 16, 16)]
                   for s in range(8)]
            addrs = [sp + (sp >> 7) * 384 for sp in sps]
            vals = [plsc.load_gather(tab_v, [a + off])
                    for a in addrs
                    for off in (0, 128, 256, 4096, 4224, 4352)]
            i = 0
            for s in range(8):
                off = tbase + (h * 8 + s) * 16
                for out_v in (out_t_v, out_r_v):
                    for c in range(3):
                        out_v[pl.ds(off + c * 128, 16)] = vals[i]
                        i += 1
        return carry

    lax.fori_loop(0, _TPW, tile_body, 0)

    obase = base * 4
    pltpu.sync_copy(out_t_v, out_t.at[pl.ds(obase, _WSTAGE)])
    pltpu.sync_copy(out_r_v, out_r.at[pl.ds(obase, _WSTAGE)])


def _tilebytes(tab_t, tab_r):
    p = jnp.pad(jnp.concatenate([tab_t, tab_r], axis=0), ((0, 0), (0, 1)))
    return p.reshape(16, 128, 4).transpose(0, 2, 1).reshape(-1)


def _untile(flat):
    tiles = flat.reshape(_TILES, 4, _LANE)
    return tiles.transpose(0, 2, 1).reshape(_TILES * _LANE, 4)[:_N, :3]


def kernel(sp_delta_t, sp_delta_r, p2sp):
    mesh = plsc.VectorSubcoreMesh(core_axis_name="c", subcore_axis_name="s")
    run = pl.kernel(
        _gather_body,
        mesh=mesh,
        compiler_params=pltpu.CompilerParams(needs_layout_passes=False),
        out_type=(
            jax.ShapeDtypeStruct((_OUT_FLAT,), jnp.float32),
            jax.ShapeDtypeStruct((_OUT_FLAT,), jnp.float32),
        ),
        scratch_types=[
            pltpu.VMEM((_NUM_SP * 8,), jnp.float32),
            pltpu.VMEM((_CHUNK,), jnp.int32),
            pltpu.VMEM((_WSTAGE,), jnp.float32),
            pltpu.VMEM((_WSTAGE,), jnp.float32),
            pltpu.SemaphoreType.DMA,
        ],
    )
    flat_t, flat_r = run(_tilebytes(sp_delta_t, sp_delta_r), p2sp)
    return _untile(flat_t), _untile(flat_r)


# speedup vs baseline: 1.0035x; 1.0028x over previous
import functools

import jax
import jax.numpy as jnp
from jax import lax
from jax.experimental import pallas as pl
from jax.experimental.pallas import tpu as pltpu
from jax.experimental.pallas import tpu_sc as plsc

_N = 100000
_NUM_SP = 1024
_LANE = 128
_TILES = 782
_OUT_FLAT = _TILES * 4 * _LANE
_NUM_CORES = 2
_NW = 32
_TPW = 25
_CHUNK = _TPW * _LANE
_WSTAGE = _TPW * 4 * _LANE
_LAST_BASE = _TILES * _LANE - _CHUNK
_LAST_VALID = _N - _LAST_BASE


def _gather_body(tab, idx, out_t, out_r,
                 tab_v, idx_v, out_t_v, out_r_v, sem):
    wid = lax.axis_index("s") * _NUM_CORES + lax.axis_index("c")
    is_tail = wid == _NW - 1
    base = jnp.where(is_tail, _LAST_BASE, wid * _CHUNK)

    ctab = pltpu.async_copy(tab, tab_v, sem)

    @pl.when(jnp.logical_not(is_tail))
    def _():
        pltpu.sync_copy(idx.at[pl.ds(base, _CHUNK)], idx_v.at[pl.ds(0, _CHUNK)])

    @pl.when(is_tail)
    def _():
        pltpu.sync_copy(idx.at[pl.ds(_LAST_BASE, _LAST_VALID)],
                        idx_v.at[pl.ds(0, _LAST_VALID)])
        zeros = jnp.zeros((16,), jnp.int32)
        for z in range(_LAST_VALID, _CHUNK, 16):
            idx_v[pl.ds(z, 16)] = zeros

    ctab.wait()

    def tile_body(t, carry):
        tbase = t * 512
        gbase = t * 128
        for h in range(1):
            sps = [idx_v[pl.ds(gbase + (h * 8 + s) * 16, 16)]
                   for s in range(8)]
            addrs = [sp + (sp >> 7) * 384 for sp in sps]
            vals = [plsc.load_gather(tab_v, [a + off])
                    for a in addrs
                    for off in (0, 128, 256, 4096, 4224, 4352)]
            i = 0
            for s in range(8):
                off = tbase + (h * 8 + s) * 16
                for out_v in (out_t_v, out_r_v):
                    for c in range(3):
                        out_v[pl.ds(off + c * 128, 16)] = vals[i]
                        i += 1
        return carry

    obase = base * 4
    h1 = 13 * 512
    h2 = _WSTAGE - h1
    lax.fori_loop(0, 13, tile_body, 0)
    c1 = pltpu.async_copy(out_t_v.at[pl.ds(0, h1)],
                          out_t.at[pl.ds(obase, h1)], sem)
    c2 = pltpu.async_copy(out_r_v.at[pl.ds(0, h1)],
                          out_r.at[pl.ds(obase, h1)], sem)
    lax.fori_loop(13, _TPW, tile_body, 0)
    c3 = pltpu.async_copy(out_t_v.at[pl.ds(h1, h2)],
                          out_t.at[pl.ds(obase + h1, h2)], sem)
    c4 = pltpu.async_copy(out_r_v.at[pl.ds(h1, h2)],
                          out_r.at[pl.ds(obase + h1, h2)], sem)
    c1.wait()
    c2.wait()
    c3.wait()
    c4.wait()


def _tilebytes(tab_t, tab_r):
    p = jnp.pad(jnp.concatenate([tab_t, tab_r], axis=0), ((0, 0), (0, 1)))
    return p.reshape(16, 128, 4).transpose(0, 2, 1).reshape(-1)


def _untile(flat):
    tiles = flat.reshape(_TILES, 4, _LANE)
    return tiles.transpose(0, 2, 1).reshape(_TILES * _LANE, 4)[:_N, :3]


def kernel(sp_delta_t, sp_delta_r, p2sp):
    mesh = plsc.VectorSubcoreMesh(core_axis_name="c", subcore_axis_name="s")
    run = pl.kernel(
        _gather_body,
        mesh=mesh,
        compiler_params=pltpu.CompilerParams(needs_layout_passes=False),
        out_type=(
            jax.ShapeDtypeStruct((_OUT_FLAT,), jnp.float32),
            jax.ShapeDtypeStruct((_OUT_FLAT,), jnp.float32),
        ),
        scratch_types=[
            pltpu.VMEM((_NUM_SP * 8,), jnp.float32),
            pltpu.VMEM((_CHUNK,), jnp.int32),
            pltpu.VMEM((_WSTAGE,), jnp.float32),
            pltpu.VMEM((_WSTAGE,), jnp.float32),
            pltpu.SemaphoreType.DMA,
        ],
    )
    flat_t, flat_r = run(_tilebytes(sp_delta_t, sp_delta_r), p2sp)
    return _untile(flat_t), _untile(flat_r)


# speedup vs baseline: 1.0038x; 1.0002x over previous
import functools

import jax
import jax.numpy as jnp
from jax import lax
from jax.experimental import pallas as pl
from jax.experimental.pallas import tpu as pltpu
from jax.experimental.pallas import tpu_sc as plsc

_N = 100000
_NUM_SP = 1024
_LANE = 128
_TILES = 782
_OUT_FLAT = _TILES * 4 * _LANE
_NUM_CORES = 2
_NW = 32
_TPW = 25
_CHUNK = _TPW * _LANE
_WSTAGE = _TPW * 4 * _LANE
_LAST_BASE = _TILES * _LANE - _CHUNK
_LAST_VALID = _N - _LAST_BASE


def _gather_body(tab, idx, out_t, out_r,
                 tab_v, idx_v, out_t_v, out_r_v, sem):
    wid = lax.axis_index("s") * _NUM_CORES + lax.axis_index("c")
    is_tail = wid == _NW - 1
    base = jnp.where(is_tail, _LAST_BASE, wid * _CHUNK)

    ctab = pltpu.async_copy(tab, tab_v, sem)

    @pl.when(jnp.logical_not(is_tail))
    def _():
        pltpu.sync_copy(idx.at[pl.ds(base, _CHUNK)], idx_v.at[pl.ds(0, _CHUNK)])

    @pl.when(is_tail)
    def _():
        pltpu.sync_copy(idx.at[pl.ds(_LAST_BASE, _LAST_VALID)],
                        idx_v.at[pl.ds(0, _LAST_VALID)])
        zeros = jnp.zeros((16,), jnp.int32)
        for z in range(_LAST_VALID, _CHUNK, 16):
            idx_v[pl.ds(z, 16)] = zeros

    ctab.wait()

    def tile_body(t, carry):
        tbase = t * 512
        gbase = t * 128
        offs = (0, 128, 256, 4096, 4224, 4352)
        sps = [idx_v[pl.ds(gbase + s * 16, 16)] for s in range(8)]
        addrs = [sp + (sp >> 7) * 384 for sp in sps]

        def emit_stores(s, vals):
            off = tbase + s * 16
            for k, (out_v, c) in enumerate(
                    ((o, c) for o in (out_t_v, out_r_v) for c in range(3))):
                out_v[pl.ds(off + c * 128, 16)] = vals[k]

        prev = None
        for s in range(8):
            vals = [plsc.load_gather(tab_v, [addrs[s] + o]) for o in offs]
            if prev is not None:
                emit_stores(s - 1, prev)
            prev = vals
        emit_stores(7, prev)
        return carry

    obase = base * 4
    h1 = 13 * 512
    h2 = _WSTAGE - h1
    lax.fori_loop(0, 13, tile_body, 0)
    c1 = pltpu.async_copy(out_t_v.at[pl.ds(0, h1)],
                          out_t.at[pl.ds(obase, h1)], sem)
    c2 = pltpu.async_copy(out_r_v.at[pl.ds(0, h1)],
                          out_r.at[pl.ds(obase, h1)], sem)
    lax.fori_loop(13, _TPW, tile_body, 0)
    c3 = pltpu.async_copy(out_t_v.at[pl.ds(h1, h2)],
                          out_t.at[pl.ds(obase + h1, h2)], sem)
    c4 = pltpu.async_copy(out_r_v.at[pl.ds(h1, h2)],
                          out_r.at[pl.ds(obase + h1, h2)], sem)
    c1.wait()
    c2.wait()
    c3.wait()
    c4.wait()


def _tilebytes(tab_t, tab_r):
    p = jnp.pad(jnp.concatenate([tab_t, tab_r], axis=0), ((0, 0), (0, 1)))
    return p.reshape(16, 128, 4).transpose(0, 2, 1).reshape(-1)


def _untile(flat):
    tiles = flat.reshape(_TILES, 4, _LANE)
    return tiles.transpose(0, 2, 1).reshape(_TILES * _LANE, 4)[:_N, :3]


def kernel(sp_delta_t, sp_delta_r, p2sp):
    mesh = plsc.VectorSubcoreMesh(core_axis_name="c", subcore_axis_name="s")
    run = pl.kernel(
        _gather_body,
        mesh=mesh,
        compiler_params=pltpu.CompilerParams(needs_layout_passes=False),
        out_type=(
            jax.ShapeDtypeStruct((_OUT_FLAT,), jnp.float32),
            jax.ShapeDtypeStruct((_OUT_FLAT,), jnp.float32),
        ),
        scratch_types=[
            pltpu.VMEM((_NUM_SP * 8,), jnp.float32),
            pltpu.VMEM((_CHUNK,), jnp.int32),
            pltpu.VMEM((_WSTAGE,), jnp.float32),
            pltpu.VMEM((_WSTAGE,), jnp.float32),
            pltpu.SemaphoreType.DMA,
        ],
    )
    flat_t, flat_r = run(_tilebytes(sp_delta_t, sp_delta_r), p2sp)
    return _untile(flat_t), _untile(flat_r)


# speedup vs baseline: 1.0115x; 1.0078x over previous
import jax
import jax.numpy as jnp
from jax import lax
from jax.experimental import pallas as pl
from jax.experimental.pallas import tpu as pltpu
from jax.experimental.pallas import tpu_sc as plsc

_N = 100000
_NUM_SP = 1024
_LANE = 128
_TILES = 782
_OUT_FLAT = _TILES * 4 * _LANE
_NUM_CORES = 2
_NW = 32
_TPW = 25
_CHUNK = _TPW * _LANE
_WSTAGE = _TPW * 4 * _LANE
_LAST_BASE = _TILES * _LANE - _CHUNK
_LAST_VALID = _N - _LAST_BASE


def _gather_body(tab, idx, out_t, out_r,
                 tab_v, idx_v, out_t_v, out_r_v, sem):
    wid = lax.axis_index("s") * _NUM_CORES + lax.axis_index("c")
    is_tail = wid == _NW - 1
    base = jnp.where(is_tail, _LAST_BASE, wid * _CHUNK)

    ctab = pltpu.async_copy(tab, tab_v, sem)

    @pl.when(jnp.logical_not(is_tail))
    def _():
        pltpu.sync_copy(idx.at[pl.ds(base, _CHUNK)], idx_v.at[pl.ds(0, _CHUNK)])

    @pl.when(is_tail)
    def _():
        pltpu.sync_copy(idx.at[pl.ds(_LAST_BASE, _LAST_VALID)],
                        idx_v.at[pl.ds(0, _LAST_VALID)])
        zeros = jnp.zeros((16,), jnp.int32)
        for z in range(_LAST_VALID, _CHUNK, 16):
            idx_v[pl.ds(z, 16)] = zeros

    ctab.wait()

    def tile_body(t, carry):
        tbase = t * 512
        gbase = t * 128
        offs = (0, 128, 256, 4096, 4224, 4352)
        sps = [idx_v[pl.ds(gbase + s * 16, 16)] for s in range(8)]
        addrs = [sp + (sp >> 7) * 384 for sp in sps]

        def emit_stores(s, vals):
            off = tbase + s * 16
            for k, (out_v, c) in enumerate(
                    ((o, c) for o in (out_t_v, out_r_v) for c in range(3))):
                out_v[pl.ds(off + c * 128, 16)] = vals[k]

        prev = None
        for s in range(8):
            vals = [plsc.load_gather(tab_v, [addrs[s] + o]) for o in offs]
            if prev is not None:
                emit_stores(s - 1, prev)
            prev = vals
        emit_stores(7, prev)
        return carry

    obase = base * 4
    h1 = 13 * 512
    h2 = _WSTAGE - h1
    lax.fori_loop(0, 13, tile_body, 0)
    c1 = pltpu.async_copy(out_t_v.at[pl.ds(0, h1)],
                          out_t.at[pl.ds(obase, h1)], sem)
    c2 = pltpu.async_copy(out_r_v.at[pl.ds(0, h1)],
                          out_r.at[pl.ds(obase, h1)], sem)
    lax.fori_loop(13, _TPW, tile_body, 0)
    c3 = pltpu.async_copy(out_t_v.at[pl.ds(h1, h2)],
                          out_t.at[pl.ds(obase + h1, h2)], sem)
    c4 = pltpu.async_copy(out_r_v.at[pl.ds(h1, h2)],
                          out_r.at[pl.ds(obase + h1, h2)], sem)
    c1.wait()
    c2.wait()
    c3.wait()
    c4.wait()


def _tilebytes(tab_t, tab_r):
    p = jnp.pad(jnp.concatenate([tab_t, tab_r], axis=0), ((0, 0), (0, 1)))
    return p.reshape(16, 128, 4).transpose(0, 2, 1).reshape(-1)


def _untile(flat):
    tiles = flat.reshape(_TILES, 4, _LANE)
    return tiles.transpose(0, 2, 1).reshape(_TILES * _LANE, 4)[:_N, :3]


def kernel(sp_delta_t, sp_delta_r, p2sp):
    mesh = plsc.VectorSubcoreMesh(core_axis_name="c", subcore_axis_name="s")
    run = pl.kernel(
        _gather_body,
        mesh=mesh,
        compiler_params=pltpu.CompilerParams(needs_layout_passes=False),
        out_type=(
            jax.ShapeDtypeStruct((_OUT_FLAT,), jnp.float32),
            jax.ShapeDtypeStruct((_OUT_FLAT,), jnp.float32),
        ),
        scratch_types=[
            pltpu.VMEM((_NUM_SP * 8,), jnp.float32),
            pltpu.VMEM((_CHUNK,), jnp.int32),
            pltpu.VMEM((_WSTAGE,), jnp.float32),
            pltpu.VMEM((_WSTAGE,), jnp.float32),
            pltpu.SemaphoreType.DMA,
        ],
    )
    flat_t, flat_r = run(_tilebytes(sp_delta_t, sp_delta_r), p2sp)
    return _untile(flat_t), _untile(flat_r)
